# baseline (speedup 1.0000x reference)
import functools

import jax
import jax.numpy as jnp
from jax import lax
from jax.experimental import pallas as pl
from jax.experimental.pallas import tpu as pltpu
from jax.experimental.pallas import tpu_sc as plsc

_N = 10000
_E = 160000
_NP = 10240
_BQ = 512
_BK = 512
_BM = 512
_DC = 128
_TILES = 16
_NB = 80
_BE = 125
_RPT = _NP // _TILES


def _flash_body(q_ref, k_ref, o_ref, m_sc, l_sc, acc_sc, *, nkb, nvalid, bk):
    j = pl.program_id(1)

    @pl.when(j == 0)
    def _():
        m_sc[...] = jnp.full_like(m_sc, -1e30)
        l_sc[...] = jnp.zeros_like(l_sc)
        acc_sc[...] = jnp.zeros_like(acc_sc)

    q = q_ref[...]
    k = k_ref[...]
    s = lax.dot_general(q, k, (((1,), (1,)), ((), ())),
                        preferred_element_type=jnp.float32, precision=lax.Precision.HIGHEST)
    col = j * bk + lax.broadcasted_iota(jnp.int32, s.shape, 1)
    s = jnp.where(col < nvalid, s, -1e30)

    m_prev = m_sc[...]
    m_new = jnp.maximum(m_prev, jnp.max(s, axis=1, keepdims=True))
    alpha = jnp.exp(m_prev - m_new)
    p = jnp.exp(s - m_new)
    l_sc[...] = l_sc[...] * alpha + jnp.sum(p, axis=1, keepdims=True)
    acc_sc[...] = acc_sc[...] * alpha + lax.dot_general(
        p, k, (((1,), (0,)), ((), ())), preferred_element_type=jnp.float32, precision=lax.Precision.HIGHEST)
    m_sc[...] = m_new

    @pl.when(j == nkb - 1)
    def _():
        o_ref[...] = q - acc_sc[...] / l_sc[...]


def _flash(q, k):
    d = q.shape[1]
    grid = (_NP // _BQ, _NP // _BK)
    return pl.pallas_call(
        functools.partial(_flash_body, nkb=grid[1], nvalid=_N, bk=_BK),
        grid=grid,
        in_specs=[pl.BlockSpec((_BQ, d), lambda i, j: (i, 0)),
                  pl.BlockSpec((_BK, d), lambda i, j: (j, 0))],
        out_specs=pl.BlockSpec((_BQ, d), lambda i, j: (i, 0)),
        out_shape=jax.ShapeDtypeStruct((_NP, d), jnp.float32),
        scratch_shapes=[pltpu.VMEM((_BQ, 1), jnp.float32),
                        pltpu.VMEM((_BQ, 1), jnp.float32),
                        pltpu.VMEM((_BQ, d), jnp.float32)],
    )(q, k)


def _mlp_body(agg_ref, mu_ref, wt_ref, wb_ref, b_ref, o_ref, hs_ref, *,
              nvalid, bm):
    i = pl.program_id(0)
    h = lax.dot_general(agg_ref[...], wt_ref[...], (((1,), (0,)), ((), ())),
                        preferred_element_type=jnp.float32, precision=lax.Precision.HIGHEST)
    h = h + lax.dot_general(mu_ref[...], wb_ref[...], (((1,), (0,)), ((), ())),
                            preferred_element_type=jnp.float32, precision=lax.Precision.HIGHEST)
    h = jnp.maximum(h + b_ref[...], 0.0)
    row = i * bm + lax.broadcasted_iota(jnp.int32, h.shape, 0)
    h = jnp.where(row < nvalid, h, 0.0)
    o_ref[...] = h

    @pl.when(i == 0)
    def _():
        hs_ref[...] = jnp.zeros_like(hs_ref)

    hs_ref[...] += jnp.sum(h, axis=0, keepdims=True)


def _mlp(agg, mu, wt, wb, b):
    d = agg.shape[1]
    hdim = wt.shape[1]
    grid = (_NP // _BM,)
    return pl.pallas_call(
        functools.partial(_mlp_body, nvalid=_N, bm=_BM),
        grid=grid,
        in_specs=[pl.BlockSpec((_BM, d), lambda i: (i, 0)),
                  pl.BlockSpec((_BM, d), lambda i: (i, 0)),
                  pl.BlockSpec((d, hdim), lambda i: (0, 0)),
                  pl.BlockSpec((d, hdim), lambda i: (0, 0)),
                  pl.BlockSpec((1, hdim), lambda i: (0, 0))],
        out_specs=[pl.BlockSpec((_BM, hdim), lambda i: (i, 0)),
                   pl.BlockSpec((1, hdim), lambda i: (0, 0))],
        out_shape=[jax.ShapeDtypeStruct((_NP, hdim), jnp.float32),
                   jax.ShapeDtypeStruct((1, hdim), jnp.float32)],
    )(agg, mu, wt, wb, b.reshape(1, hdim))


def _readout_body(hl_ref, hr_ref, w3t_ref, w3b_ref, b3_ref, w4_ref, b4_ref,
                  o_ref):
    h = lax.dot_general(hl_ref[...], w3t_ref[...], (((1,), (0,)), ((), ())),
                        preferred_element_type=jnp.float32, precision=lax.Precision.HIGHEST)
    h = h + lax.dot_general(hr_ref[...], w3b_ref[...], (((1,), (0,)), ((), ())),
                            preferred_element_type=jnp.float32, precision=lax.Precision.HIGHEST)
    h = jnp.maximum(h + b3_ref[...], 0.0)
    o_ref[...] = lax.dot_general(h, w4_ref[...], (((1,), (0,)), ((), ())),
                                 preferred_element_type=jnp.float32, precision=lax.Precision.HIGHEST) + b4_ref[...]


def _readout(hl, hr, w3t, w3b, b3, w4, b4):
    hdim = w3t.shape[1]
    return pl.pallas_call(
        _readout_body,
        out_shape=jax.ShapeDtypeStruct((1, 1), jnp.float32),
    )(hl, hr, w3t, w3b, b3.reshape(1, hdim), w4, b4.reshape(1, 1))


def _sc_segsum_body(ch_per_core, xl_hbm, xr_hbm, el_hbm, er_hbm, z_hbm,
                    outl_hbm, outr_hbm, src_v, dst_v, rows_v, agg_sh, sem):
    cid = lax.axis_index("c")
    sid = lax.axis_index("s")
    row0 = sid * _RPT
    for side in range(2):
        x_hbm = (xl_hbm, xr_hbm)[side]
        e_hbm = (el_hbm, er_hbm)[side]
        out_hbm = (outl_hbm, outr_hbm)[side]
        for u in range(ch_per_core):
            chunk = cid * ch_per_core + u
            pltpu.sync_copy(z_hbm, agg_sh.at[pl.ds(row0, _RPT)])
            pltpu.sync_copy(e_hbm.at[0, sid], src_v)
            pltpu.sync_copy(e_hbm.at[1, sid], dst_v)
            plsc.subcore_barrier()

            def batch(bi, _):
                pltpu.async_copy(x_hbm.at[chunk].at[src_v.at[bi]], rows_v,
                                 sem).wait()
                pltpu.sync_copy(rows_v, agg_sh.at[dst_v.at[bi]], add=True)
                return 0

            lax.fori_loop(0, _NB, batch, 0)
            plsc.subcore_barrier()
            pltpu.sync_copy(agg_sh.at[pl.ds(row0, _RPT)],
                            out_hbm.at[chunk, pl.ds(row0, _RPT)])
            plsc.subcore_barrier()


def _sc_segsum(xl_c, xr_c, el, er, zeros):
    nchunks = xl_c.shape[0]
    mesh = plsc.VectorSubcoreMesh(core_axis_name="c", subcore_axis_name="s")
    out_t = (jax.ShapeDtypeStruct((nchunks, _NP, _DC), jnp.float32),
             jax.ShapeDtypeStruct((nchunks, _NP, _DC), jnp.float32))
    k = pl.kernel(
        functools.partial(_sc_segsum_body, nchunks // 2),
        out_type=out_t,
        mesh=mesh,
        scratch_types=[
            pltpu.VMEM((_NB, _BE), jnp.int32),
            pltpu.VMEM((_NB, _BE), jnp.int32),
            pltpu.VMEM((_BE, _DC), jnp.float32),
            pltpu.VMEM_SHARED((_NP, _DC), jnp.float32),
            pltpu.SemaphoreType.DMA,
        ],
    )
    return k(xl_c, xr_c, el, er, zeros)


def _chunked(x, nchunks):
    return x.reshape(_NP, nchunks, _DC).transpose(1, 0, 2)


def _unchunk(x_c, d):
    return x_c.transpose(1, 0, 2).reshape(_NP, d)


def kernel(xl, edge_index_l, xr, edge_index_r,
           W0, b0, W1, b1, W2, b2, W3, b3, W4, b4):
    f32 = jnp.float32
    xl_p = jnp.pad(xl.astype(f32), ((0, _NP - _N), (0, 0)))
    xr_p = jnp.pad(xr.astype(f32), ((0, _NP - _N), (0, 0)))
    el = edge_index_l.astype(jnp.int32).reshape(2, _TILES, _NB, _BE)
    er = edge_index_r.astype(jnp.int32).reshape(2, _TILES, _NB, _BE)
    zeros = jnp.zeros((_RPT, _DC), f32)

    hl = hr = None
    for (W, b) in ((W0, b0), (W1, b1), (W2, b2)):
        d = xl_p.shape[1]
        nchunks = d // _DC
        mu_rl = _flash(xl_p, xr_p)
        mu_lr = _flash(xr_p, xl_p)
        aggl_c, aggr_c = _sc_segsum(_chunked(xl_p, nchunks),
                                    _chunked(xr_p, nchunks), el, er, zeros)
        aggl = _unchunk(aggl_c, d)
        aggr = _unchunk(aggr_c, d)
        wt, wb = W[:d], W[d:]
        xl_p, hl = _mlp(aggl, mu_rl, wt, wb, b)
        xr_p, hr = _mlp(aggr, mu_lr, wt, wb, b)

    hdim = W3.shape[1]
    return _readout(hl, hr, W3[:hdim], W3[hdim:], b3, W4, b4)


# speedup vs baseline: 2.1628x; 2.1628x over previous
import functools

import jax
import jax.numpy as jnp
from jax import lax
from jax.experimental import pallas as pl
from jax.experimental.pallas import tpu as pltpu
from jax.experimental.pallas import tpu_sc as plsc

_N = 10000
_E = 160000
_NP = 10240
_BQ = 512
_BK = 512
_BM = 512
_DC = 128
_TILES = 16
_NB = 80
_BE = 125
_RPT = _NP // _TILES


def _dflash_body(q_ref, k_ref, or_ref, ocT_ref, mr, lr, accr, mc, lc, accc,
                 *, nib, nkb, nvalid, bq, bk):
    i = pl.program_id(0)
    j = pl.program_id(1)
    q = q_ref[...]
    k = k_ref[...]
    s = lax.dot_general(q, k, (((1,), (1,)), ((), ())),
                        preferred_element_type=jnp.float32,
                        precision=lax.Precision.HIGHEST)
    row = i * bq + lax.broadcasted_iota(jnp.int32, s.shape, 0)
    col = j * bk + lax.broadcasted_iota(jnp.int32, s.shape, 1)
    s_r = jnp.where(col < nvalid, s, -1e30)
    s_c = jnp.where(row < nvalid, s, -1e30)

    @pl.when(j == 0)
    def _():
        mr[...] = jnp.full_like(mr, -1e30)
        lr[...] = jnp.zeros_like(lr)
        accr[...] = jnp.zeros_like(accr)

    m_prev = mr[...]
    m_new = jnp.maximum(m_prev, jnp.max(s_r, axis=1, keepdims=True))
    alpha = jnp.exp(m_prev - m_new)
    p = jnp.exp(s_r - m_new)
    lr[...] = lr[...] * alpha + jnp.sum(p, axis=1, keepdims=True)
    accr[...] = accr[...] * alpha + lax.dot_general(
        p, k, (((1,), (0,)), ((), ())), preferred_element_type=jnp.float32)
    mr[...] = m_new

    @pl.when(j == nkb - 1)
    def _():
        or_ref[...] = q - accr[...] / lr[...]

    csl = pl.ds(j * bk, bk)
    cmax = jnp.max(s_c, axis=0, keepdims=True)
    first = i == 0
    m_prev_c = jnp.where(first, jnp.full_like(cmax, -1e30),
                         mc[pl.ds(j, 1)].reshape(1, bk))
    m_new_c = jnp.maximum(m_prev_c, cmax)
    alpha_c = jnp.exp(m_prev_c - m_new_c)
    p_c = jnp.exp(s_c - m_new_c)
    l_prev_c = jnp.where(first, 0.0, lc[pl.ds(j, 1)].reshape(1, bk))
    lc[pl.ds(j, 1)] = (l_prev_c * alpha_c
                       + jnp.sum(p_c, axis=0, keepdims=True)).reshape(1, 1, bk)
    acc_prev = jnp.where(first, 0.0, accc[:, csl])
    accc[:, csl] = acc_prev * alpha_c + lax.dot_general(
        q, p_c, (((0,), (0,)), ((), ())), preferred_element_type=jnp.float32)
    mc[pl.ds(j, 1)] = m_new_c.reshape(1, 1, bk)

    @pl.when(i == nib - 1)
    def _():
        ocT_ref[...] = accc[:, csl] / lc[pl.ds(j, 1)].reshape(1, bk)


def _dflash(q, k):
    d = q.shape[1]
    nib, nkb = _NP // _BQ, _NP // _BK
    return pl.pallas_call(
        functools.partial(_dflash_body, nib=nib, nkb=nkb, nvalid=_N,
                          bq=_BQ, bk=_BK),
        grid=(nib, nkb),
        in_specs=[pl.BlockSpec((_BQ, d), lambda i, j: (i, 0)),
                  pl.BlockSpec((_BK, d), lambda i, j: (j, 0))],
        out_specs=[
            pl.BlockSpec((_BQ, d), lambda i, j: (i, 0)),
            pl.BlockSpec((d, _BK),
                         lambda i, j: (0, jnp.where(i == nib - 1, j, 0))),
        ],
        out_shape=[jax.ShapeDtypeStruct((_NP, d), jnp.float32),
                   jax.ShapeDtypeStruct((d, _NP), jnp.float32)],
        scratch_shapes=[pltpu.VMEM((_BQ, 1), jnp.float32),
                        pltpu.VMEM((_BQ, 1), jnp.float32),
                        pltpu.VMEM((_BQ, d), jnp.float32),
                        pltpu.VMEM((nkb, 1, _BK), jnp.float32),
                        pltpu.VMEM((nkb, 1, _BK), jnp.float32),
                        pltpu.VMEM((d, _NP), jnp.float32)],
    )(q, k)


def _mlp_body(agg_ref, mu_ref, wt_ref, wb_ref, b_ref, o_ref, hs_ref, *,
              nvalid, bm):
    i = pl.program_id(0)
    h = lax.dot_general(agg_ref[...], wt_ref[...], (((1,), (0,)), ((), ())),
                        preferred_element_type=jnp.float32)
    h = h + lax.dot_general(mu_ref[...], wb_ref[...], (((1,), (0,)), ((), ())),
                            preferred_element_type=jnp.float32)
    h = jnp.maximum(h + b_ref[...], 0.0)
    row = i * bm + lax.broadcasted_iota(jnp.int32, h.shape, 0)
    h = jnp.where(row < nvalid, h, 0.0)
    o_ref[...] = h

    @pl.when(i == 0)
    def _():
        hs_ref[...] = jnp.zeros_like(hs_ref)

    hs_ref[...] += jnp.sum(h, axis=0, keepdims=True)


def _mlp(agg, mu, wt, wb, b):
    d = agg.shape[1]
    hdim = wt.shape[1]
    grid = (_NP // _BM,)
    return pl.pallas_call(
        functools.partial(_mlp_body, nvalid=_N, bm=_BM),
        grid=grid,
        in_specs=[pl.BlockSpec((_BM, d), lambda i: (i, 0)),
                  pl.BlockSpec((_BM, d), lambda i: (i, 0)),
                  pl.BlockSpec((d, hdim), lambda i: (0, 0)),
                  pl.BlockSpec((d, hdim), lambda i: (0, 0)),
                  pl.BlockSpec((1, hdim), lambda i: (0, 0))],
        out_specs=[pl.BlockSpec((_BM, hdim), lambda i: (i, 0)),
                   pl.BlockSpec((1, hdim), lambda i: (0, 0))],
        out_shape=[jax.ShapeDtypeStruct((_NP, hdim), jnp.float32),
                   jax.ShapeDtypeStruct((1, hdim), jnp.float32)],
    )(agg, mu, wt, wb, b.reshape(1, hdim))


def _readout_body(hl_ref, hr_ref, w3t_ref, w3b_ref, b3_ref, w4_ref, b4_ref,
                  o_ref):
    h = lax.dot_general(hl_ref[...], w3t_ref[...], (((1,), (0,)), ((), ())),
                        preferred_element_type=jnp.float32, precision=lax.Precision.HIGHEST)
    h = h + lax.dot_general(hr_ref[...], w3b_ref[...], (((1,), (0,)), ((), ())),
                            preferred_element_type=jnp.float32, precision=lax.Precision.HIGHEST)
    h = jnp.maximum(h + b3_ref[...], 0.0)
    o_ref[...] = lax.dot_general(h, w4_ref[...], (((1,), (0,)), ((), ())),
                                 preferred_element_type=jnp.float32, precision=lax.Precision.HIGHEST) + b4_ref[...]


def _readout(hl, hr, w3t, w3b, b3, w4, b4):
    hdim = w3t.shape[1]
    return pl.pallas_call(
        _readout_body,
        out_shape=jax.ShapeDtypeStruct((1, 1), jnp.float32),
    )(hl, hr, w3t, w3b, b3.reshape(1, hdim), w4, b4.reshape(1, 1))


def _sc_segsum_body(ch_per_core, xl_hbm, xr_hbm, el_hbm, er_hbm, z_hbm,
                    outl_hbm, outr_hbm, src_v, dst_v, rows_v, agg_sh, sem):
    cid = lax.axis_index("c")
    sid = lax.axis_index("s")
    row0 = sid * _RPT
    for side in range(2):
        x_hbm = (xl_hbm, xr_hbm)[side]
        e_hbm = (el_hbm, er_hbm)[side]
        out_hbm = (outl_hbm, outr_hbm)[side]
        for u in range(ch_per_core):
            chunk = cid * ch_per_core + u
            pltpu.sync_copy(z_hbm, agg_sh.at[pl.ds(row0, _RPT)])
            pltpu.sync_copy(e_hbm.at[0, sid], src_v)
            pltpu.sync_copy(e_hbm.at[1, sid], dst_v)
            plsc.subcore_barrier()

            def batch(bi, _):
                pltpu.async_copy(x_hbm.at[chunk].at[src_v.at[bi]], rows_v,
                                 sem).wait()
                pltpu.sync_copy(rows_v, agg_sh.at[dst_v.at[bi]], add=True)
                return 0

            lax.fori_loop(0, _NB, batch, 0)
            plsc.subcore_barrier()
            pltpu.sync_copy(agg_sh.at[pl.ds(row0, _RPT)],
                            out_hbm.at[chunk, pl.ds(row0, _RPT)])
            plsc.subcore_barrier()


def _sc_segsum(xl_c, xr_c, el, er, zeros):
    nchunks = xl_c.shape[0]
    mesh = plsc.VectorSubcoreMesh(core_axis_name="c", subcore_axis_name="s")
    out_t = (jax.ShapeDtypeStruct((nchunks, _NP, _DC), jnp.float32),
             jax.ShapeDtypeStruct((nchunks, _NP, _DC), jnp.float32))
    k = pl.kernel(
        functools.partial(_sc_segsum_body, nchunks // 2),
        out_type=out_t,
        mesh=mesh,
        scratch_types=[
            pltpu.VMEM((_NB, _BE), jnp.int32),
            pltpu.VMEM((_NB, _BE), jnp.int32),
            pltpu.VMEM((_BE, _DC), jnp.float32),
            pltpu.VMEM_SHARED((_NP, _DC), jnp.float32),
            pltpu.SemaphoreType.DMA,
        ],
    )
    return k(xl_c, xr_c, el, er, zeros)


def _chunked(x, nchunks):
    return x.reshape(_NP, nchunks, _DC).transpose(1, 0, 2)


def _unchunk(x_c, d):
    return x_c.transpose(1, 0, 2).reshape(_NP, d)


def kernel(xl, edge_index_l, xr, edge_index_r,
           W0, b0, W1, b1, W2, b2, W3, b3, W4, b4):
    f32 = jnp.float32
    xl_p = jnp.pad(xl.astype(f32), ((0, _NP - _N), (0, 0)))
    xr_p = jnp.pad(xr.astype(f32), ((0, _NP - _N), (0, 0)))
    el = edge_index_l.astype(jnp.int32).reshape(2, _TILES, _NB, _BE)
    er = edge_index_r.astype(jnp.int32).reshape(2, _TILES, _NB, _BE)
    zeros = jnp.zeros((_RPT, _DC), f32)

    hl = hr = None
    for (W, b) in ((W0, b0), (W1, b1), (W2, b2)):
        d = xl_p.shape[1]
        nchunks = d // _DC
        mu_rl, attT_lr = _dflash(xl_p, xr_p)
        mu_lr = xr_p - attT_lr.T
        aggl_c, aggr_c = _sc_segsum(_chunked(xl_p, nchunks),
                                    _chunked(xr_p, nchunks), el, er, zeros)
        aggl = _unchunk(aggl_c, d)
        aggr = _unchunk(aggr_c, d)
        wt, wb = W[:d], W[d:]
        xl_p, hl = _mlp(aggl, mu_rl, wt, wb, b)
        xr_p, hr = _mlp(aggr, mu_lr, wt, wb, b)

    hdim = W3.shape[1]
    return _readout(hl, hr, W3[:hdim], W3[hdim:], b3, W4, b4)


# speedup vs baseline: 2.6303x; 1.2161x over previous
import functools

import jax
import jax.numpy as jnp
from jax import lax
from jax.experimental import pallas as pl
from jax.experimental.pallas import tpu as pltpu
from jax.experimental.pallas import tpu_sc as plsc

_N = 10000
_E = 160000
_NP = 10240
_BQ = 512
_BK = 512
_BM = 512
_DC = 128
_TILES = 16
_NB = 80
_BE = 125
_RPT = _NP // _TILES


def _dflash_body(q_ref, k_ref, or_ref, ocT_ref, mr, lr, accr, mc, lc, accc,
                 *, nib, nkb, nvalid, bq, bk):
    i = pl.program_id(0)
    j = pl.program_id(1)
    q = q_ref[...]
    k = k_ref[...]
    qh = q.astype(jnp.bfloat16)
    ql = (q - qh.astype(jnp.float32)).astype(jnp.bfloat16)
    kh = k.astype(jnp.bfloat16)
    kl = (k - kh.astype(jnp.float32)).astype(jnp.bfloat16)
    dn = (((1,), (1,)), ((), ()))
    s = (lax.dot_general(qh, kh, dn, preferred_element_type=jnp.float32)
         + (lax.dot_general(qh, kl, dn, preferred_element_type=jnp.float32)
            + lax.dot_general(ql, kh, dn, preferred_element_type=jnp.float32)))
    row = i * bq + lax.broadcasted_iota(jnp.int32, s.shape, 0)
    col = j * bk + lax.broadcasted_iota(jnp.int32, s.shape, 1)
    s_r = jnp.where(col < nvalid, s, -1e30)
    s_c = jnp.where(row < nvalid, s, -1e30)

    @pl.when(j == 0)
    def _():
        mr[...] = jnp.full_like(mr, -1e30)
        lr[...] = jnp.zeros_like(lr)
        accr[...] = jnp.zeros_like(accr)

    m_prev = mr[...]
    m_new = jnp.maximum(m_prev, jnp.max(s_r, axis=1, keepdims=True))
    alpha = jnp.exp(m_prev - m_new)
    p = jnp.exp(s_r - m_new)
    lr[...] = lr[...] * alpha + jnp.sum(p, axis=1, keepdims=True)
    accr[...] = accr[...] * alpha + lax.dot_general(
        p, k, (((1,), (0,)), ((), ())), preferred_element_type=jnp.float32)
    mr[...] = m_new

    @pl.when(j == nkb - 1)
    def _():
        or_ref[...] = q - accr[...] / lr[...]

    csl = pl.ds(j * bk, bk)
    cmax = jnp.max(s_c, axis=0, keepdims=True)
    first = i == 0
    m_prev_c = jnp.where(first, jnp.full_like(cmax, -1e30),
                         mc[pl.ds(j, 1)].reshape(1, bk))
    m_new_c = jnp.maximum(m_prev_c, cmax)
    alpha_c = jnp.exp(m_prev_c - m_new_c)
    p_c = jnp.exp(s_c - m_new_c)
    l_prev_c = jnp.where(first, 0.0, lc[pl.ds(j, 1)].reshape(1, bk))
    lc[pl.ds(j, 1)] = (l_prev_c * alpha_c
                       + jnp.sum(p_c, axis=0, keepdims=True)).reshape(1, 1, bk)
    acc_prev = jnp.where(first, 0.0, accc[:, csl])
    accc[:, csl] = acc_prev * alpha_c + lax.dot_general(
        q, p_c, (((0,), (0,)), ((), ())), preferred_element_type=jnp.float32)
    mc[pl.ds(j, 1)] = m_new_c.reshape(1, 1, bk)

    @pl.when(i == nib - 1)
    def _():
        ocT_ref[...] = accc[:, csl] / lc[pl.ds(j, 1)].reshape(1, bk)


def _dflash(q, k):
    d = q.shape[1]
    nib, nkb = _NP // _BQ, _NP // _BK
    return pl.pallas_call(
        functools.partial(_dflash_body, nib=nib, nkb=nkb, nvalid=_N,
                          bq=_BQ, bk=_BK),
        grid=(nib, nkb),
        in_specs=[pl.BlockSpec((_BQ, d), lambda i, j: (i, 0)),
                  pl.BlockSpec((_BK, d), lambda i, j: (j, 0))],
        out_specs=[
            pl.BlockSpec((_BQ, d), lambda i, j: (i, 0)),
            pl.BlockSpec((d, _BK),
                         lambda i, j: (0, jnp.where(i == nib - 1, j, 0))),
        ],
        out_shape=[jax.ShapeDtypeStruct((_NP, d), jnp.float32),
                   jax.ShapeDtypeStruct((d, _NP), jnp.float32)],
        scratch_shapes=[pltpu.VMEM((_BQ, 1), jnp.float32),
                        pltpu.VMEM((_BQ, 1), jnp.float32),
                        pltpu.VMEM((_BQ, d), jnp.float32),
                        pltpu.VMEM((nkb, 1, _BK), jnp.float32),
                        pltpu.VMEM((nkb, 1, _BK), jnp.float32),
                        pltpu.VMEM((d, _NP), jnp.float32)],
    )(q, k)


def _mlp_body(agg_ref, mu_ref, wt_ref, wb_ref, b_ref, o_ref, hs_ref, *,
              nvalid, bm):
    i = pl.program_id(0)
    h = lax.dot_general(agg_ref[...], wt_ref[...], (((1,), (0,)), ((), ())),
                        preferred_element_type=jnp.float32)
    h = h + lax.dot_general(mu_ref[...], wb_ref[...], (((1,), (0,)), ((), ())),
                            preferred_element_type=jnp.float32)
    h = jnp.maximum(h + b_ref[...], 0.0)
    row = i * bm + lax.broadcasted_iota(jnp.int32, h.shape, 0)
    h = jnp.where(row < nvalid, h, 0.0)
    o_ref[...] = h

    @pl.when(i == 0)
    def _():
        hs_ref[...] = jnp.zeros_like(hs_ref)

    hs_ref[...] += jnp.sum(h, axis=0, keepdims=True)


def _mlp(agg, mu, wt, wb, b):
    d = agg.shape[1]
    hdim = wt.shape[1]
    grid = (_NP // _BM,)
    return pl.pallas_call(
        functools.partial(_mlp_body, nvalid=_N, bm=_BM),
        grid=grid,
        in_specs=[pl.BlockSpec((_BM, d), lambda i: (i, 0)),
                  pl.BlockSpec((_BM, d), lambda i: (i, 0)),
                  pl.BlockSpec((d, hdim), lambda i: (0, 0)),
                  pl.BlockSpec((d, hdim), lambda i: (0, 0)),
                  pl.BlockSpec((1, hdim), lambda i: (0, 0))],
        out_specs=[pl.BlockSpec((_BM, hdim), lambda i: (i, 0)),
                   pl.BlockSpec((1, hdim), lambda i: (0, 0))],
        out_shape=[jax.ShapeDtypeStruct((_NP, hdim), jnp.float32),
                   jax.ShapeDtypeStruct((1, hdim), jnp.float32)],
    )(agg, mu, wt, wb, b.reshape(1, hdim))


def _readout_body(hl_ref, hr_ref, w3t_ref, w3b_ref, b3_ref, w4_ref, b4_ref,
                  o_ref):
    h = lax.dot_general(hl_ref[...], w3t_ref[...], (((1,), (0,)), ((), ())),
                        preferred_element_type=jnp.float32, precision=lax.Precision.HIGHEST)
    h = h + lax.dot_general(hr_ref[...], w3b_ref[...], (((1,), (0,)), ((), ())),
                            preferred_element_type=jnp.float32, precision=lax.Precision.HIGHEST)
    h = jnp.maximum(h + b3_ref[...], 0.0)
    o_ref[...] = lax.dot_general(h, w4_ref[...], (((1,), (0,)), ((), ())),
                                 preferred_element_type=jnp.float32, precision=lax.Precision.HIGHEST) + b4_ref[...]


def _readout(hl, hr, w3t, w3b, b3, w4, b4):
    hdim = w3t.shape[1]
    return pl.pallas_call(
        _readout_body,
        out_shape=jax.ShapeDtypeStruct((1, 1), jnp.float32),
    )(hl, hr, w3t, w3b, b3.reshape(1, hdim), w4, b4.reshape(1, 1))


def _sc_segsum_body(ch_per_core, xl_hbm, xr_hbm, el_hbm, er_hbm, z_hbm,
                    outl_hbm, outr_hbm, src_v, dst_v, rows_v, agg_sh, sem):
    cid = lax.axis_index("c")
    sid = lax.axis_index("s")
    row0 = sid * _RPT
    for side in range(2):
        x_hbm = (xl_hbm, xr_hbm)[side]
        e_hbm = (el_hbm, er_hbm)[side]
        out_hbm = (outl_hbm, outr_hbm)[side]
        for u in range(ch_per_core):
            chunk = cid * ch_per_core + u
            pltpu.sync_copy(z_hbm, agg_sh.at[pl.ds(row0, _RPT)])
            pltpu.sync_copy(e_hbm.at[0, sid], src_v)
            pltpu.sync_copy(e_hbm.at[1, sid], dst_v)
            plsc.subcore_barrier()

            def batch(bi, _):
                pltpu.async_copy(x_hbm.at[chunk].at[src_v.at[bi]], rows_v,
                                 sem).wait()
                pltpu.sync_copy(rows_v, agg_sh.at[dst_v.at[bi]], add=True)
                return 0

            lax.fori_loop(0, _NB, batch, 0)
            plsc.subcore_barrier()
            pltpu.sync_copy(agg_sh.at[pl.ds(row0, _RPT)],
                            out_hbm.at[chunk, pl.ds(row0, _RPT)])
            plsc.subcore_barrier()


def _sc_segsum(xl_c, xr_c, el, er, zeros):
    nchunks = xl_c.shape[0]
    mesh = plsc.VectorSubcoreMesh(core_axis_name="c", subcore_axis_name="s")
    out_t = (jax.ShapeDtypeStruct((nchunks, _NP, _DC), jnp.float32),
             jax.ShapeDtypeStruct((nchunks, _NP, _DC), jnp.float32))
    k = pl.kernel(
        functools.partial(_sc_segsum_body, nchunks // 2),
        out_type=out_t,
        mesh=mesh,
        scratch_types=[
            pltpu.VMEM((_NB, _BE), jnp.int32),
            pltpu.VMEM((_NB, _BE), jnp.int32),
            pltpu.VMEM((_BE, _DC), jnp.float32),
            pltpu.VMEM_SHARED((_NP, _DC), jnp.float32),
            pltpu.SemaphoreType.DMA,
        ],
    )
    return k(xl_c, xr_c, el, er, zeros)


def _chunked(x, nchunks):
    return x.reshape(_NP, nchunks, _DC).transpose(1, 0, 2)


def _unchunk(x_c, d):
    return x_c.transpose(1, 0, 2).reshape(_NP, d)


def kernel(xl, edge_index_l, xr, edge_index_r,
           W0, b0, W1, b1, W2, b2, W3, b3, W4, b4):
    f32 = jnp.float32
    xl_p = jnp.pad(xl.astype(f32), ((0, _NP - _N), (0, 0)))
    xr_p = jnp.pad(xr.astype(f32), ((0, _NP - _N), (0, 0)))
    el = edge_index_l.astype(jnp.int32).reshape(2, _TILES, _NB, _BE)
    er = edge_index_r.astype(jnp.int32).reshape(2, _TILES, _NB, _BE)
    zeros = jnp.zeros((_RPT, _DC), f32)

    hl = hr = None
    for (W, b) in ((W0, b0), (W1, b1), (W2, b2)):
        d = xl_p.shape[1]
        nchunks = d // _DC
        mu_rl, attT_lr = _dflash(xl_p, xr_p)
        mu_lr = xr_p - attT_lr.T
        aggl_c, aggr_c = _sc_segsum(_chunked(xl_p, nchunks),
                                    _chunked(xr_p, nchunks), el, er, zeros)
        aggl = _unchunk(aggl_c, d)
        aggr = _unchunk(aggr_c, d)
        wt, wb = W[:d], W[d:]
        xl_p, hl = _mlp(aggl, mu_rl, wt, wb, b)
        xr_p, hr = _mlp(aggr, mu_lr, wt, wb, b)

    hdim = W3.shape[1]
    return _readout(hl, hr, W3[:hdim], W3[hdim:], b3, W4, b4)
